# baseline (speedup 1.0000x reference)
import functools

import jax
import jax.numpy as jnp
import numpy as np
from jax import lax
from jax.experimental import pallas as pl
from jax.experimental.pallas import tpu as pltpu
from jax.experimental.pallas import tpu_sc as plsc

NUM_PERTS = 100000
LATENT_DIM = 256
BATCH = 16384

_NC, _NS = 2, 16
_NW = _NC * _NS
_CH = 128
_HW = LATENT_DIM // 2

_GRP = np.arange(_HW) // 16
_LANE = np.arange(_HW) % 16
_C_LO = (32 * _GRP + _LANE).astype(np.int32)
_C_HI = (32 * _GRP + 16 + _LANE).astype(np.int32)


def _sc_gather_pack(pert_idx, table, nrows):
    mesh = plsc.VectorSubcoreMesh(core_axis_name="c", subcore_axis_name="s")
    bpw = nrows // _NW
    nch = max(1, bpw // _CH)
    ch = min(bpw, _CH)
    ngrp = LATENT_DIM // 32

    @functools.partial(
        pl.kernel,
        mesh=mesh,
        out_type=jax.ShapeDtypeStruct((nrows, _HW), jnp.uint32),
        scratch_types=[
            pltpu.VMEM((nch, ch), jnp.int32),
            pltpu.VMEM((ch, LATENT_DIM), jnp.float32),
            pltpu.VMEM((ch, LATENT_DIM), jnp.float32),
            pltpu.VMEM((ch, _HW), jnp.uint32),
            pltpu.VMEM((ch, _HW), jnp.uint32),
            pltpu.SemaphoreType.DMA,
            pltpu.SemaphoreType.DMA,
            pltpu.SemaphoreType.DMA,
            pltpu.SemaphoreType.DMA,
        ],
    )
    def gather_k(idx_hbm, table_hbm, out_hbm, idx_v, f0, f1, u0, u1,
                 sg0, sg1, so0, so1):
        wid = lax.axis_index("s") * _NC + lax.axis_index("c")
        base = wid * bpw
        pltpu.sync_copy(idx_hbm.at[pl.ds(wid * nch, nch)], idx_v)
        fbufs, ubufs = (f0, f1), (u0, u1)
        gsems, osems = (sg0, sg1), (so0, so1)
        gcp = [None, None]
        ocp = [None, None]
        gcp[0] = pltpu.async_copy(
            table_hbm.at[idx_v.at[0]], f0, sg0)

        def convert(fbuf, ubuf):
            fb = fbuf.bitcast(jnp.uint32)

            @plsc.parallel_loop(0, ch, unroll=4)
            def _(r):
                for g in range(ngrp):
                    a = fb[r, pl.ds(g * 32, 16)]
                    bvec = fb[r, pl.ds(g * 32 + 16, 16)]
                    ubuf[r, pl.ds(g * 16, 16)] = (
                        (a >> jnp.uint32(16))
                        | (bvec & jnp.uint32(0xFFFF0000)))

        for c in range(nch):
            if c + 1 < nch:
                gcp[(c + 1) % 2] = pltpu.async_copy(
                    table_hbm.at[idx_v.at[c + 1]],
                    fbufs[(c + 1) % 2], gsems[(c + 1) % 2])
            gcp[c % 2].wait()
            if ocp[c % 2] is not None:
                ocp[c % 2].wait()
            convert(fbufs[c % 2], ubufs[c % 2])
            ocp[c % 2] = pltpu.async_copy(
                ubufs[c % 2], out_hbm.at[pl.ds(base + c * ch, ch)],
                osems[c % 2])
        for c in range(min(2, nch)):
            if ocp[c] is not None:
                ocp[c].wait()

    return gather_k(pert_idx, table)


def _tc_body(x_ref, wlo_ref, whi_ref, b_ref, g_ref, bt_ref, o_ref):
    u = x_ref[...]
    xlo = lax.bitcast_convert_type(u << jnp.uint32(16), jnp.float32)
    xhi = lax.bitcast_convert_type(u & jnp.uint32(0xFFFF0000), jnp.float32)
    h = jnp.dot(xlo, wlo_ref[...], preferred_element_type=jnp.float32)
    h = h + jnp.dot(xhi, whi_ref[...], preferred_element_type=jnp.float32)
    h = h + b_ref[...]
    mean = jnp.mean(h, axis=1, keepdims=True)
    cen = h - mean
    var = jnp.mean(cen * cen, axis=1, keepdims=True)
    xhat = cen * lax.rsqrt(var + 1e-5)
    h2 = xhat * g_ref[...] + bt_ref[...]
    o_ref[...] = 0.5 * h2 * (1.0 + lax.erf(h2 * 0.7071067811865476))


def _tc_body_carry(x_ref, wlo_ref, whi_ref, b_ref, g_ref, bt_ref, prev_ref,
                   o_ref):
    del prev_ref
    _tc_body(x_ref, wlo_ref, whi_ref, b_ref, g_ref, bt_ref, o_ref)


def _tc_mlp_chunk(emb, Wlo, Whi, b, gamma, beta, prev, row_off, bm):
    nb = emb.shape[0] // bm
    off_b = row_off // bm
    in_specs = [
        pl.BlockSpec((bm, _HW), lambda i: (i, 0)),
        pl.BlockSpec((_HW, LATENT_DIM), lambda i: (0, 0)),
        pl.BlockSpec((_HW, LATENT_DIM), lambda i: (0, 0)),
        pl.BlockSpec((1, LATENT_DIM), lambda i: (0, 0)),
        pl.BlockSpec((1, LATENT_DIM), lambda i: (0, 0)),
        pl.BlockSpec((1, LATENT_DIM), lambda i: (0, 0)),
    ]
    args = [emb, Wlo, Whi, b.reshape(1, LATENT_DIM),
            gamma.reshape(1, LATENT_DIM), beta.reshape(1, LATENT_DIM)]
    kwargs = {}
    body = _tc_body
    if prev is not None:
        in_specs.append(pl.BlockSpec(memory_space=pl.ANY))
        args.append(prev)
        kwargs["input_output_aliases"] = {6: 0}
        body = _tc_body_carry
    return pl.pallas_call(
        body,
        grid=(nb,),
        in_specs=in_specs,
        out_specs=pl.BlockSpec((bm, LATENT_DIM),
                               lambda i, _o=off_b: (i + _o, 0)),
        out_shape=jax.ShapeDtypeStruct((BATCH, LATENT_DIM), jnp.float32),
        **kwargs,
    )(*args)


_NCHUNK = 2
_CB = BATCH // _NCHUNK
_BM = 1024


def kernel(pert_idx, table, W, b, gamma, beta):
    idx = pert_idx.astype(jnp.int32)
    Wlo = W[_C_LO, :]
    Whi = W[_C_HI, :]
    embs = [_sc_gather_pack(
        lax.slice(idx, (c * _CB,), ((c + 1) * _CB,)).reshape(_CB // _CH, _CH),
        table, _CB) for c in range(_NCHUNK)]
    out = None
    for c in range(_NCHUNK):
        out = _tc_mlp_chunk(embs[c], Wlo, Whi, b, gamma, beta, out,
                            c * _CB, _BM)
    return out


# speedup vs baseline: 1.0382x; 1.0382x over previous
import functools

import jax
import jax.numpy as jnp
import numpy as np
from jax import lax
from jax.experimental import pallas as pl
from jax.experimental.pallas import tpu as pltpu
from jax.experimental.pallas import tpu_sc as plsc

NUM_PERTS = 100000
LATENT_DIM = 256
BATCH = 16384

_NC, _NS = 2, 16
_NW = _NC * _NS
_CH = 128
_HW = LATENT_DIM // 2

_GRP = np.arange(_HW) // 16
_LANE = np.arange(_HW) % 16
_C_LO = (32 * _GRP + _LANE).astype(np.int32)
_C_HI = (32 * _GRP + 16 + _LANE).astype(np.int32)


def _sc_gather_pack(pert_idx, table, nrows):
    mesh = plsc.VectorSubcoreMesh(core_axis_name="c", subcore_axis_name="s")
    bpw = nrows // _NW
    nch = max(1, bpw // _CH)
    ch = min(bpw, _CH)
    ngrp = LATENT_DIM // 32

    @functools.partial(
        pl.kernel,
        mesh=mesh,
        out_type=jax.ShapeDtypeStruct((nrows, _HW), jnp.uint32),
        scratch_types=[
            pltpu.VMEM((nch, ch), jnp.int32),
            pltpu.VMEM((ch, LATENT_DIM), jnp.float32),
            pltpu.VMEM((ch, LATENT_DIM), jnp.float32),
            pltpu.VMEM((ch, _HW), jnp.uint32),
            pltpu.VMEM((ch, _HW), jnp.uint32),
            pltpu.SemaphoreType.DMA,
            pltpu.SemaphoreType.DMA,
            pltpu.SemaphoreType.DMA,
            pltpu.SemaphoreType.DMA,
        ],
    )
    def gather_k(idx_hbm, table_hbm, out_hbm, idx_v, f0, f1, u0, u1,
                 sg0, sg1, so0, so1):
        wid = lax.axis_index("s") * _NC + lax.axis_index("c")
        base = wid * bpw
        pltpu.sync_copy(idx_hbm.at[pl.ds(wid * nch, nch)], idx_v)
        fbufs, ubufs = (f0, f1), (u0, u1)
        gsems, osems = (sg0, sg1), (so0, so1)
        gcp = [None, None]
        ocp = [None, None]
        gcp[0] = pltpu.async_copy(
            table_hbm.at[idx_v.at[0]], f0, sg0)

        def convert(fbuf, ubuf):
            fb = fbuf.bitcast(jnp.uint32)

            @plsc.parallel_loop(0, ch, unroll=4)
            def _(r):
                for g in range(ngrp):
                    a = fb[r, pl.ds(g * 32, 16)]
                    bvec = fb[r, pl.ds(g * 32 + 16, 16)]
                    ubuf[r, pl.ds(g * 16, 16)] = (
                        (a >> jnp.uint32(16))
                        | (bvec & jnp.uint32(0xFFFF0000)))

        for c in range(nch):
            if c + 1 < nch:
                gcp[(c + 1) % 2] = pltpu.async_copy(
                    table_hbm.at[idx_v.at[c + 1]],
                    fbufs[(c + 1) % 2], gsems[(c + 1) % 2])
            gcp[c % 2].wait()
            if ocp[c % 2] is not None:
                ocp[c % 2].wait()
            convert(fbufs[c % 2], ubufs[c % 2])
            ocp[c % 2] = pltpu.async_copy(
                ubufs[c % 2], out_hbm.at[pl.ds(base + c * ch, ch)],
                osems[c % 2])
        for c in range(min(2, nch)):
            if ocp[c] is not None:
                ocp[c].wait()

    return gather_k(pert_idx, table)


def _tc_body(x_ref, wlo_ref, whi_ref, b_ref, g_ref, bt_ref, o_ref):
    u = x_ref[...]
    xlo = lax.bitcast_convert_type(u << jnp.uint32(16), jnp.float32)
    xhi = lax.bitcast_convert_type(u & jnp.uint32(0xFFFF0000), jnp.float32)
    h = jnp.dot(xlo, wlo_ref[...], preferred_element_type=jnp.float32)
    h = h + jnp.dot(xhi, whi_ref[...], preferred_element_type=jnp.float32)
    h = h + b_ref[...]
    mean = jnp.mean(h, axis=1, keepdims=True)
    cen = h - mean
    var = jnp.mean(cen * cen, axis=1, keepdims=True)
    xhat = cen * lax.rsqrt(var + 1e-5)
    h2 = xhat * g_ref[...] + bt_ref[...]
    o_ref[...] = 0.5 * h2 * (1.0 + lax.erf(h2 * 0.7071067811865476))


def _tc_body_carry(x_ref, wlo_ref, whi_ref, b_ref, g_ref, bt_ref, prev_ref,
                   o_ref):
    del prev_ref
    _tc_body(x_ref, wlo_ref, whi_ref, b_ref, g_ref, bt_ref, o_ref)


def _tc_mlp_chunk(emb, Wlo, Whi, b, gamma, beta, prev, row_off, bm):
    nb = emb.shape[0] // bm
    off_b = row_off // bm
    in_specs = [
        pl.BlockSpec((bm, _HW), lambda i: (i, 0)),
        pl.BlockSpec((_HW, LATENT_DIM), lambda i: (0, 0)),
        pl.BlockSpec((_HW, LATENT_DIM), lambda i: (0, 0)),
        pl.BlockSpec((1, LATENT_DIM), lambda i: (0, 0)),
        pl.BlockSpec((1, LATENT_DIM), lambda i: (0, 0)),
        pl.BlockSpec((1, LATENT_DIM), lambda i: (0, 0)),
    ]
    args = [emb, Wlo, Whi, b.reshape(1, LATENT_DIM),
            gamma.reshape(1, LATENT_DIM), beta.reshape(1, LATENT_DIM)]
    kwargs = {}
    body = _tc_body
    if prev is not None:
        in_specs.append(pl.BlockSpec(memory_space=pl.ANY))
        args.append(prev)
        kwargs["input_output_aliases"] = {6: 0}
        body = _tc_body_carry
    return pl.pallas_call(
        body,
        grid=(nb,),
        in_specs=in_specs,
        out_specs=pl.BlockSpec((bm, LATENT_DIM),
                               lambda i, _o=off_b: (i + _o, 0)),
        out_shape=jax.ShapeDtypeStruct((BATCH, LATENT_DIM), jnp.float32),
        **kwargs,
    )(*args)


_NCHUNK = 2
_CB = BATCH // _NCHUNK
_BM = 2048


def kernel(pert_idx, table, W, b, gamma, beta):
    idx = pert_idx.astype(jnp.int32)
    Wlo = W[_C_LO, :]
    Whi = W[_C_HI, :]
    embs = [_sc_gather_pack(
        lax.slice(idx, (c * _CB,), ((c + 1) * _CB,)).reshape(_CB // _CH, _CH),
        table, _CB) for c in range(_NCHUNK)]
    out = None
    for c in range(_NCHUNK):
        out = _tc_mlp_chunk(embs[c], Wlo, Whi, b, gamma, beta, out,
                            c * _CB, _BM)
    return out


# speedup vs baseline: 1.0556x; 1.0168x over previous
import functools

import jax
import jax.numpy as jnp
from jax import lax
from jax.experimental import pallas as pl
from jax.experimental.pallas import tpu as pltpu
from jax.experimental.pallas import tpu_sc as plsc

NUM_PERTS = 100000
LATENT_DIM = 256
BATCH = 16384

_NC, _NS = 2, 16
_NW = _NC * _NS
_CH = 128


def _sc_gather(pert_idx, table, nrows):
    mesh = plsc.VectorSubcoreMesh(core_axis_name="c", subcore_axis_name="s")
    bpw = nrows // _NW
    nch = max(1, bpw // _CH)
    ch = min(bpw, _CH)

    @functools.partial(
        pl.kernel,
        mesh=mesh,
        out_type=jax.ShapeDtypeStruct((nrows, LATENT_DIM), jnp.float32),
        scratch_types=[
            pltpu.VMEM((bpw,), jnp.int32),
            pltpu.VMEM((ch, LATENT_DIM), jnp.float32),
            pltpu.VMEM((ch, LATENT_DIM), jnp.float32),
            pltpu.SemaphoreType.DMA,
            pltpu.SemaphoreType.DMA,
        ],
    )
    def gather_k(idx_hbm, table_hbm, out_hbm, idx_v, buf0, buf1, sem0, sem1):
        wid = lax.axis_index("s") * _NC + lax.axis_index("c")
        base = wid * bpw
        pltpu.sync_copy(idx_hbm.at[pl.ds(base, bpw)], idx_v)
        bufs = (buf0, buf1)
        sems = (sem0, sem1)
        cps = [None, None]
        cps[0] = pltpu.async_copy(
            table_hbm.at[idx_v.at[pl.ds(0, ch)]], buf0, sem0)
        for c in range(nch):
            if c + 1 < nch:
                cps[(c + 1) % 2] = pltpu.async_copy(
                    table_hbm.at[idx_v.at[pl.ds((c + 1) * ch, ch)]],
                    bufs[(c + 1) % 2], sems[(c + 1) % 2])
            cps[c % 2].wait()
            pltpu.sync_copy(bufs[c % 2], out_hbm.at[pl.ds(base + c * ch, ch)])

    return gather_k(pert_idx, table)


def _tc_body(x_ref, w_ref, b_ref, g_ref, bt_ref, o_ref):
    x = x_ref[...]
    h = jnp.dot(x, w_ref[...], preferred_element_type=jnp.float32)
    h = h + b_ref[...]
    mean = jnp.mean(h, axis=1, keepdims=True)
    cen = h - mean
    var = jnp.mean(cen * cen, axis=1, keepdims=True)
    xhat = cen * lax.rsqrt(var + 1e-5)
    h2 = xhat * g_ref[...] + bt_ref[...]
    o_ref[...] = 0.5 * h2 * (1.0 + lax.erf(h2 * 0.7071067811865476))


def _tc_body_carry(x_ref, w_ref, b_ref, g_ref, bt_ref, prev_ref, o_ref):
    del prev_ref
    _tc_body(x_ref, w_ref, b_ref, g_ref, bt_ref, o_ref)


def _tc_mlp_chunk(emb, W, b, gamma, beta, prev, row_off, bm):
    nb = emb.shape[0] // bm
    off_b = row_off // bm
    in_specs = [
        pl.BlockSpec((bm, LATENT_DIM), lambda i: (i, 0)),
        pl.BlockSpec((LATENT_DIM, LATENT_DIM), lambda i: (0, 0)),
        pl.BlockSpec((1, LATENT_DIM), lambda i: (0, 0)),
        pl.BlockSpec((1, LATENT_DIM), lambda i: (0, 0)),
        pl.BlockSpec((1, LATENT_DIM), lambda i: (0, 0)),
    ]
    args = [emb, W, b.reshape(1, LATENT_DIM), gamma.reshape(1, LATENT_DIM),
            beta.reshape(1, LATENT_DIM)]
    kwargs = {}
    body = _tc_body
    if prev is not None:
        in_specs.append(pl.BlockSpec(memory_space=pl.ANY))
        args.append(prev)
        kwargs["input_output_aliases"] = {5: 0}
        body = _tc_body_carry
    return pl.pallas_call(
        body,
        grid=(nb,),
        in_specs=in_specs,
        out_specs=pl.BlockSpec((bm, LATENT_DIM),
                               lambda i, _o=off_b: (i + _o, 0)),
        out_shape=jax.ShapeDtypeStruct((BATCH, LATENT_DIM), jnp.float32),
        **kwargs,
    )(*args)


_NCHUNK = 2
_CB = BATCH // _NCHUNK
_BM = 2048


def kernel(pert_idx, table, W, b, gamma, beta):
    idx = pert_idx.astype(jnp.int32)
    embs = [_sc_gather(lax.slice(idx, (c * _CB,), ((c + 1) * _CB,)), table,
                       _CB) for c in range(_NCHUNK)]
    out = None
    for c in range(_NCHUNK):
        out = _tc_mlp_chunk(embs[c], W, b, gamma, beta, out, c * _CB, _BM)
    return out
